# baseline (speedup 1.0000x reference)
import functools

import jax
import jax.numpy as jnp
from jax import lax
from jax.experimental import pallas as pl
from jax.experimental.pallas import tpu as pltpu
from jax.experimental.pallas import tpu_sc as plsc

BATCH = 4096
HIST = 200
DIM = 32
TOTAL = BATCH * HIST
TILE = 128
NBUF = 4


def _make_gather():
    info = plsc.get_sparse_core_info()
    nc, ns = info.num_cores, info.num_subcores
    nw = nc * ns
    n_tiles = TOTAL // TILE
    per_w = n_tiles // nw
    idx_per_w = per_w * TILE
    n_groups = per_w // NBUF
    c2_n = BATCH // TILE

    mesh = plsc.VectorSubcoreMesh(core_axis_name="c", subcore_axis_name="s")

    @functools.partial(
        pl.kernel,
        mesh=mesh,
        out_type=jax.ShapeDtypeStruct((HIST, 4, c2_n, 8 * TILE), jnp.float32),
        scratch_types=[
            pltpu.VMEM((idx_per_w,), jnp.int32),
            [pltpu.VMEM((TILE, DIM), jnp.float32) for _ in range(NBUF)],
            [pltpu.VMEM((DIM * TILE,), jnp.float32) for _ in range(NBUF)],
            [pltpu.SemaphoreType.DMA for _ in range(NBUF)],
            [pltpu.SemaphoreType.DMA for _ in range(NBUF)],
        ],
        compiler_params=pltpu.CompilerParams(
            use_tc_tiling_on_sc=False, needs_layout_passes=False
        ),
    )
    def gather_kernel(idx_hbm, table_hbm, out_hbm, idx_v, gbufs, tbufs, gsems, ssems):
        wid = lax.axis_index("s") * nc + lax.axis_index("c")
        base_t = wid * per_w
        pltpu.sync_copy(idx_hbm.at[pl.ds(base_t * TILE, idx_per_w)], idx_v)

        lane = lax.iota(jnp.int32, 16)
        zero = lane * 0

        def start_gather(t, b):
            pltpu.async_copy(
                table_hbm.at[idx_v.at[pl.ds(t * TILE, TILE)]],
                gbufs[b],
                gsems[b],
            )

        def wait_gather(t, b):
            pltpu.make_async_copy(
                table_hbm.at[idx_v.at[pl.ds(t * TILE, TILE)]],
                gbufs[b],
                gsems[b],
            ).wait()

        def transpose(b):
            def drow(d, carry):
                for k in range(TILE // 16):
                    src = plsc.load_gather(gbufs[b], [lane + k * 16, zero + d])
                    tbufs[b][pl.ds(d * TILE + k * 16, 16)] = src
                return carry

            lax.fori_loop(0, DIM, drow, 0)

        def start_store(t, b):
            gt = base_t + t
            h = gt // c2_n
            c2 = gt % c2_n
            for dr in range(4):
                pltpu.async_copy(
                    tbufs[b].at[pl.ds(dr * 8 * TILE, 8 * TILE)],
                    out_hbm.at[h, dr, c2],
                    ssems[b],
                )

        def wait_store(t, b):
            gt = base_t + t
            h = gt // c2_n
            c2 = gt % c2_n
            for dr in range(4):
                pltpu.make_async_copy(
                    tbufs[b].at[pl.ds(dr * 8 * TILE, 8 * TILE)],
                    out_hbm.at[h, dr, c2],
                    ssems[b],
                ).wait()

        for b in range(NBUF):
            start_gather(b, b)

        for b in range(NBUF):
            wait_gather(b, b)
            transpose(b)
            start_gather(b + NBUF, b)
            start_store(b, b)

        def body(j, carry):
            for b in range(NBUF):
                t = j * NBUF + b
                wait_gather(t, b)
                wait_store(t - NBUF, b)
                transpose(b)
                start_gather(t + NBUF, b)
                start_store(t, b)
            return carry

        lax.fori_loop(1, n_groups - 1, body, 0)

        for b in range(NBUF):
            t = (n_groups - 1) * NBUF + b
            wait_gather(t, b)
            wait_store(t - NBUF, b)
            transpose(b)
            start_store(t, b)

        for b in range(NBUF):
            t = (n_groups - 1) * NBUF + b
            wait_store(t, b)

    return gather_kernel


_gather = _make_gather()


@jax.jit
def kernel(xs, table):
    idx_flat = xs.T.reshape(-1)
    out5 = _gather(idx_flat, table)
    out = out5.reshape(HIST, 4, BATCH // TILE, 8, TILE)
    out = out.transpose(2, 4, 0, 1, 3).reshape(BATCH, HIST, DIM)
    return out
